# Initial kernel scaffold; baseline (speedup 1.0000x reference)
import functools

import jax
import jax.numpy as jnp
from jax import lax
from jax.experimental import pallas as pl
from jax.experimental.pallas import tpu as pltpu
from jax.experimental.pallas import tpu_sc as plsc

_PEOPLE = 30
_JOINTS = 17
_KP = _PEOPLE * _JOINTS
_KP_PAD = 512
_KP_WORDS_PAD = 1024
_EPS = 1e-6


@functools.lru_cache(maxsize=None)
def _build(num_images, tags_per_image):
    mesh = plsc.VectorSubcoreMesh(core_axis_name="c", subcore_axis_name="s")

    def body(tags_ref, kp_ref, out_ref, kp_v, idx_v, vis_v, gat_v,
             mean_v, val_v, row_v, sem):
        wid = lax.axis_index("s") * 2 + lax.axis_index("c")

        @pl.when(wid < num_images)
        def _():
            img = wid
            pltpu.sync_copy(kp_ref.at[img], kp_v)

            lanes = lax.iota(jnp.int32, 16)

            for c in range(_KP_PAD // 16):
                flat = lanes + c * 16
                a_idx = jnp.minimum(flat * 2, 2 * _KP - 2)
                a_vis = jnp.minimum(flat * 2 + 1, 2 * _KP - 1)
                kv = plsc.load_gather(kp_v, [a_idx])
                vv = plsc.load_gather(kp_v, [a_vis])
                idx_v[pl.ds(c * 16, 16)] = kv + img * tags_per_image
                vis_v[pl.ds(c * 16, 16)] = (vv > 0).astype(jnp.float32)

            copies = []
            for b in range(_KP_PAD // 128):
                copies.append(pltpu.async_copy(
                    tags_ref.at[idx_v.at[pl.ds(b * 128, 128)]],
                    gat_v.at[pl.ds(b * 128, 128)], sem))
            for cp in copies:
                cp.wait()

            pull_acc = jnp.zeros((16,), jnp.float32)
            nval_acc = jnp.zeros((16,), jnp.float32)
            for pv in range(2):
                p0 = lanes + pv * 16
                pmask = p0 < _PEOPLE
                s = jnp.zeros((16,), jnp.float32)
                cnt = jnp.zeros((16,), jnp.float32)
                for j in range(_JOINTS):
                    addr = p0 * _JOINTS + j
                    g = plsc.load_gather(gat_v, [addr])
                    vi = plsc.load_gather(vis_v, [addr])
                    vb = (vi > 0.0) & pmask
                    s = s + jnp.where(vb, g, 0.0)
                    cnt = cnt + jnp.where(vb, 1.0, 0.0)
                safe = jnp.maximum(cnt, 1.0)
                mean = s / safe
                valid = cnt > 0.0
                d2s = jnp.zeros((16,), jnp.float32)
                for j in range(_JOINTS):
                    addr = p0 * _JOINTS + j
                    g = plsc.load_gather(gat_v, [addr])
                    vi = plsc.load_gather(vis_v, [addr])
                    vb = (vi > 0.0) & pmask
                    d = g - mean
                    d2s = d2s + jnp.where(vb, d * d, 0.0)
                pull_acc = pull_acc + jnp.where(valid, d2s / safe, 0.0)
                nval_acc = nval_acc + jnp.where(valid, 1.0, 0.0)
                mean_v[pl.ds(pv * 16, 16)] = mean
                val_v[pl.ds(pv * 16, 16)] = jnp.where(valid, 1.0, 0.0)

            push_acc = jnp.zeros((16,), jnp.float32)
            for q in range(1, _PEOPLE):
                qs = jnp.full((16,), q, jnp.int32)
                mq = plsc.load_gather(mean_v, [qs])
                vq = plsc.load_gather(val_v, [qs])
                for pv in range(2):
                    if pv * 16 < q:
                        p0 = lanes + pv * 16
                        mp = mean_v[pl.ds(pv * 16, 16)]
                        vp = val_v[pl.ds(pv * 16, 16)]
                        d = mp - mq
                        d2 = d * d
                        sel = ((p0 < q) & (d2 != 0.0)
                               & (vp > 0.0) & (vq > 0.0))
                        push_acc = push_acc + jnp.where(
                            sel, jnp.exp(-d2), 0.0)

            pull = jnp.sum(pull_acc)
            push = jnp.sum(push_acc)
            n = jnp.sum(nval_acc)
            push_o = jnp.where(n > 0.0, push / ((n - 1.0) * n + _EPS), 0.0)
            pull_o = jnp.where(n > 0.0, pull / (n + _EPS), 0.0)
            row_v[...] = jnp.where(lanes == 0, push_o,
                                   jnp.where(lanes == 1, pull_o, 0.0))
            pltpu.sync_copy(row_v, out_ref.at[img])

    return pl.kernel(
        body,
        out_type=jax.ShapeDtypeStruct((num_images, 16), jnp.float32),
        mesh=mesh,
        scratch_types=[
            pltpu.VMEM((_KP_WORDS_PAD,), jnp.int32),
            pltpu.VMEM((_KP_PAD,), jnp.int32),
            pltpu.VMEM((34 * 16,), jnp.float32),
            pltpu.VMEM((34 * 16,), jnp.float32),
            pltpu.VMEM((32,), jnp.float32),
            pltpu.VMEM((32,), jnp.float32),
            pltpu.VMEM((16,), jnp.float32),
            pltpu.SemaphoreType.DMA,
        ],
    )


def kernel(tags, keypoints):
    num_images, tags_per_image, _ = tags.shape
    tags_flat = tags.reshape(num_images * tags_per_image)
    kp_flat = keypoints.reshape(num_images, _PEOPLE * _JOINTS * 2)
    kp_pad = jnp.pad(
        kp_flat, ((0, 0), (0, _KP_WORDS_PAD - kp_flat.shape[1])))
    out = _build(num_images, tags_per_image)(tags_flat, kp_pad)
    return out[:, :2]


# speedup vs baseline: 10.8655x; 10.8655x over previous
import functools

import jax
import jax.numpy as jnp
from jax import lax
from jax.experimental import pallas as pl
from jax.experimental.pallas import tpu as pltpu
from jax.experimental.pallas import tpu_sc as plsc

_PEOPLE = 30
_JOINTS = 17
_KP = _PEOPLE * _JOINTS
_KP_PAD = 512
_KP_WORDS_PAD = 1024
_EPS = 1e-6


@functools.lru_cache(maxsize=None)
def _build(num_images, tags_per_image):
    mesh = plsc.VectorSubcoreMesh(core_axis_name="c", subcore_axis_name="s")

    def body(tags_ref, kp_ref, out_ref, kp_v, idx_v, vis_v, gat_v,
             mean_v, val_v, row_v, sem):
        wid = lax.axis_index("s") * 2 + lax.axis_index("c")

        @pl.when(wid < num_images)
        def _():
            img = wid
            pltpu.sync_copy(kp_ref.at[img], kp_v)

            lanes = lax.iota(jnp.int32, 16)

            for c in range(_KP_PAD // 16):
                flat = lanes + c * 16
                a_idx = jnp.minimum(flat * 2, 2 * _KP - 2)
                a_vis = jnp.minimum(flat * 2 + 1, 2 * _KP - 1)
                kv = plsc.load_gather(kp_v, [a_idx])
                vv = plsc.load_gather(kp_v, [a_vis])
                idx_v[pl.ds(c * 16, 16)] = kv + img * tags_per_image
                vis_v[pl.ds(c * 16, 16)] = (vv > 0).astype(jnp.float32)

            copies = []
            for b in range(_KP_PAD // 128):
                copies.append(pltpu.async_copy(
                    tags_ref.at[idx_v.at[pl.ds(b * 128, 128)]],
                    gat_v.at[pl.ds(b * 128, 128)], sem))
            for cp in copies:
                cp.wait()

            pull_acc = jnp.zeros((16,), jnp.float32)
            nval_acc = jnp.zeros((16,), jnp.float32)
            for pv in range(2):
                p0 = lanes + pv * 16
                pmask = p0 < _PEOPLE
                s = jnp.zeros((16,), jnp.float32)
                cnt = jnp.zeros((16,), jnp.float32)
                for j in range(_JOINTS):
                    addr = p0 * _JOINTS + j
                    g = plsc.load_gather(gat_v, [addr])
                    vi = plsc.load_gather(vis_v, [addr])
                    vb = (vi > 0.0) & pmask
                    s = s + jnp.where(vb, g, 0.0)
                    cnt = cnt + jnp.where(vb, 1.0, 0.0)
                safe = jnp.maximum(cnt, 1.0)
                mean = s / safe
                valid = cnt > 0.0
                d2s = jnp.zeros((16,), jnp.float32)
                for j in range(_JOINTS):
                    addr = p0 * _JOINTS + j
                    g = plsc.load_gather(gat_v, [addr])
                    vi = plsc.load_gather(vis_v, [addr])
                    vb = (vi > 0.0) & pmask
                    d = g - mean
                    d2s = d2s + jnp.where(vb, d * d, 0.0)
                pull_acc = pull_acc + jnp.where(valid, d2s / safe, 0.0)
                nval_acc = nval_acc + jnp.where(valid, 1.0, 0.0)
                mean_v[pl.ds(pv * 16, 16)] = mean
                val_v[pl.ds(pv * 16, 16)] = jnp.where(valid, 1.0, 0.0)

            push_acc = jnp.zeros((16,), jnp.float32)
            for q in range(1, _PEOPLE):
                qs = jnp.full((16,), q, jnp.int32)
                mq = plsc.load_gather(mean_v, [qs])
                vq = plsc.load_gather(val_v, [qs])
                for pv in range(2):
                    if pv * 16 < q:
                        p0 = lanes + pv * 16
                        mp = mean_v[pl.ds(pv * 16, 16)]
                        vp = val_v[pl.ds(pv * 16, 16)]
                        d = mp - mq
                        d2 = d * d
                        sel = ((p0 < q) & (d2 != 0.0)
                               & (vp > 0.0) & (vq > 0.0))
                        push_acc = push_acc + jnp.where(
                            sel, jnp.exp(-d2), 0.0)

            pull = jnp.broadcast_to(jnp.sum(pull_acc), (16,))
            push = jnp.broadcast_to(jnp.sum(push_acc), (16,))
            n = jnp.broadcast_to(jnp.sum(nval_acc), (16,))
            push_o = jnp.where(n > 0.0, push / ((n - 1.0) * n + _EPS), 0.0)
            pull_o = jnp.where(n > 0.0, pull / (n + _EPS), 0.0)
            row_v[...] = jnp.where(lanes == 0, push_o,
                                   jnp.where(lanes == 1, pull_o, 0.0))
            pltpu.sync_copy(row_v, out_ref.at[img])

    return pl.kernel(
        body,
        out_type=jax.ShapeDtypeStruct((num_images, 16), jnp.float32),
        mesh=mesh,
        compiler_params=pltpu.CompilerParams(needs_layout_passes=False),
        scratch_types=[
            pltpu.VMEM((_KP_WORDS_PAD,), jnp.int32),
            pltpu.VMEM((_KP_PAD,), jnp.int32),
            pltpu.VMEM((34 * 16,), jnp.float32),
            pltpu.VMEM((34 * 16,), jnp.float32),
            pltpu.VMEM((32,), jnp.float32),
            pltpu.VMEM((32,), jnp.float32),
            pltpu.VMEM((16,), jnp.float32),
            pltpu.SemaphoreType.DMA,
        ],
    )


def kernel(tags, keypoints):
    num_images, tags_per_image, _ = tags.shape
    tags_flat = tags.reshape(num_images * tags_per_image)
    kp_flat = keypoints.reshape(num_images, _PEOPLE * _JOINTS * 2)
    kp_pad = jnp.pad(
        kp_flat, ((0, 0), (0, _KP_WORDS_PAD - kp_flat.shape[1])))
    out = _build(num_images, tags_per_image)(tags_flat, kp_pad)
    return out[:, :2]
